# Initial kernel scaffold; baseline (speedup 1.0000x reference)
import jax
import jax.numpy as jnp
from jax.experimental import pallas as pl


def kernel(a_x, a_vec, m_x, a2a_edge_index, a2m_edge_index, m2a_edge_index, a2a_edge_weights, a2m_edge_weights, m2a_edge_weights, a2a_edge_attr, a2m_edge_attr, m2a_edge_attr, a2a_edge_vecs, W_s1, W_s2, W_e, W_a2m, W_m2a, Wq, Wk, Wv, Wo, ln_s_g, ln_s_b, ln_f_g, ln_f_b, ln_l_g, ln_l_b):
    raise NotImplementedError("write your pallas kernel here")


# speedup vs baseline: 4.4366x; 4.4366x over previous
import functools

import jax
import jax.numpy as jnp
from jax import lax
from jax.experimental import pallas as pl
from jax.experimental.pallas import tpu as pltpu
from jax.experimental.pallas import tpu_sc as plsc

H = 128
N = 10000
M = 8192
NG = 512
NH = 8
E = 160000
EAM = 80000
E_PAD = 163840
EAM_PAD = 81920
N_ACC = N + 16


def _ln_body(x_ref, g_ref, b_ref, o_ref, *, scale):
    x = x_ref[...]
    mu = jnp.mean(x, axis=-1, keepdims=True)
    var = jnp.mean((x - mu) ** 2, axis=-1, keepdims=True)
    o_ref[...] = ((x - mu) / jnp.sqrt(var + 1e-5) * g_ref[...] + b_ref[...]) * scale


def _ln(x, g, b, scale=1.0, blk=512):
    n = x.shape[0]
    grid = (n + blk - 1) // blk
    return pl.pallas_call(
        functools.partial(_ln_body, scale=scale),
        grid=(grid,),
        in_specs=[pl.BlockSpec((blk, H), lambda i: (i, 0)),
                  pl.BlockSpec((1, H), lambda i: (0, 0)),
                  pl.BlockSpec((1, H), lambda i: (0, 0))],
        out_specs=pl.BlockSpec((blk, H), lambda i: (i, 0)),
        out_shape=jax.ShapeDtypeStruct((n, H), jnp.float32),
    )(x, g.reshape(1, H), b.reshape(1, H))


def _edge_body(gs_ref, gd_ref, gv_ref, en_ref, w_ref, ev_ref, ea_ref,
               w1_ref, w2_ref, we_ref,
               v1_ref, se_ref, v20_ref, v21_ref, v22_ref):
    en = en_ref[...]
    w = w_ref[...]
    msg = gs_ref[...] * en * w
    v1_ref[...] = jnp.dot(msg, w1_ref[...], preferred_element_type=jnp.float32)
    gate = jnp.dot(msg, w2_ref[...], preferred_element_type=jnp.float32)
    se = jnp.dot(gs_ref[...] * gd_ref[...], we_ref[...],
                 preferred_element_type=jnp.float32)
    se_ref[...] = se + ea_ref[...]
    for c, vo in enumerate((v20_ref, v21_ref, v22_ref)):
        vo[...] = (gv_ref[:, c * H:(c + 1) * H] * en
                   + ev_ref[:, c:c + 1] * gate) * w


def _edge_math(gs, gd, gv, en, w2d, ev, ea, W_s1, W_s2, W_e, blk=1024):
    grid = pl.cdiv(E, blk)
    outs = pl.pallas_call(
        _edge_body,
        grid=(grid,),
        in_specs=[pl.BlockSpec((blk, H), lambda i: (i, 0)),
                  pl.BlockSpec((blk, H), lambda i: (i, 0)),
                  pl.BlockSpec((blk, 3 * H), lambda i: (i, 0)),
                  pl.BlockSpec((blk, H), lambda i: (i, 0)),
                  pl.BlockSpec((blk, 1), lambda i: (i, 0)),
                  pl.BlockSpec((blk, 3), lambda i: (i, 0)),
                  pl.BlockSpec((blk, H), lambda i: (i, 0)),
                  pl.BlockSpec((H, H), lambda i: (0, 0)),
                  pl.BlockSpec((H, H), lambda i: (0, 0)),
                  pl.BlockSpec((H, H), lambda i: (0, 0))],
        out_specs=[pl.BlockSpec((blk, H), lambda i: (i, 0)),
                   pl.BlockSpec((blk, H), lambda i: (i, 0)),
                   pl.BlockSpec((blk, H), lambda i: (i, 0)),
                   pl.BlockSpec((blk, H), lambda i: (i, 0)),
                   pl.BlockSpec((blk, H), lambda i: (i, 0))],
        out_shape=[jax.ShapeDtypeStruct((E_PAD, H), jnp.float32),
                   jax.ShapeDtypeStruct((E, H), jnp.float32),
                   jax.ShapeDtypeStruct((E_PAD, H), jnp.float32),
                   jax.ShapeDtypeStruct((E_PAD, H), jnp.float32),
                   jax.ShapeDtypeStruct((E_PAD, H), jnp.float32)],
    )(gs, gd, gv, en, w2d, ev, ea, W_s1, W_s2, W_e)
    return outs


def _pair_body(g_ref, attr_ref, w_ref, wm_ref, o_ref):
    o_ref[...] = jnp.dot(g_ref[...] * attr_ref[...] * w_ref[...], wm_ref[...],
                         preferred_element_type=jnp.float32)


def _pair_math(g, attr, w2d, Wm, nedge, npad, blk=1024):
    grid = pl.cdiv(nedge, blk)
    return pl.pallas_call(
        _pair_body,
        grid=(grid,),
        in_specs=[pl.BlockSpec((blk, H), lambda i: (i, 0)),
                  pl.BlockSpec((blk, H), lambda i: (i, 0)),
                  pl.BlockSpec((blk, 1), lambda i: (i, 0)),
                  pl.BlockSpec((H, H), lambda i: (0, 0))],
        out_specs=pl.BlockSpec((blk, H), lambda i: (i, 0)),
        out_shape=jax.ShapeDtypeStruct((npad, H), jnp.float32),
    )(g, attr, w2d, Wm)


def _mha_body(x_ref, wq_ref, wk_ref, wv_ref, wo_ref, mx_ref, o_ref):
    x = x_ref[...]
    q = jnp.dot(x, wq_ref[...], preferred_element_type=jnp.float32)
    k = jnp.dot(x, wk_ref[...], preferred_element_type=jnp.float32)
    v = jnp.dot(x, wv_ref[...], preferred_element_type=jnp.float32)
    hd = H // NH
    outs = []
    for h in range(NH):
        qh = q[:, h * hd:(h + 1) * hd]
        kh = k[:, h * hd:(h + 1) * hd]
        vh = v[:, h * hd:(h + 1) * hd]
        att = lax.dot_general(qh, kh, (((1,), (1,)), ((), ())),
                              preferred_element_type=jnp.float32) * (1.0 / 4.0)
        att = jax.nn.softmax(att, axis=-1)
        outs.append(jnp.dot(att, vh, preferred_element_type=jnp.float32))
    o = jnp.concatenate(outs, axis=1)
    o_ref[...] = (jnp.dot(o, wo_ref[...], preferred_element_type=jnp.float32)
                  - mx_ref[...])


def _mha(m_xn, Wq, Wk, Wv, Wo, m_x):
    grid = M // NG
    return pl.pallas_call(
        _mha_body,
        grid=(grid,),
        in_specs=[pl.BlockSpec((NG, H), lambda i: (i, 0)),
                  pl.BlockSpec((H, H), lambda i: (0, 0)),
                  pl.BlockSpec((H, H), lambda i: (0, 0)),
                  pl.BlockSpec((H, H), lambda i: (0, 0)),
                  pl.BlockSpec((H, H), lambda i: (0, 0)),
                  pl.BlockSpec((NG, H), lambda i: (i, 0))],
        out_specs=pl.BlockSpec((NG, H), lambda i: (i, 0)),
        out_shape=jax.ShapeDtypeStruct((M, H), jnp.float32),
    )(m_xn, Wq, Wk, Wv, Wo, m_x)


def _comb_body(p_ref, r_ref, o_ref, *, sign):
    o_ref[...] = p_ref[0] + p_ref[1] + sign * r_ref[...]


def _combine(parts, res, sign, blk=512):
    n, d = res.shape
    grid = (n + blk - 1) // blk
    return pl.pallas_call(
        functools.partial(_comb_body, sign=sign),
        grid=(grid,),
        in_specs=[pl.BlockSpec((2, blk, d), lambda i: (0, i, 0)),
                  pl.BlockSpec((blk, d), lambda i: (i, 0))],
        out_specs=pl.BlockSpec((blk, d), lambda i: (i, 0)),
        out_shape=jax.ShapeDtypeStruct((n, d), jnp.float32),
    )(parts, res)


def _combv_body(p_ref, r_ref, o_ref):
    for c in range(3):
        o_ref[:, c * H:(c + 1) * H] = (p_ref[c, 0] + p_ref[c, 1]
                                       - r_ref[:, c * H:(c + 1) * H])


def _combine_vec(pv, av384, blk=512):
    grid = (N + blk - 1) // blk
    return pl.pallas_call(
        _combv_body,
        grid=(grid,),
        in_specs=[pl.BlockSpec((3, 2, blk, H), lambda i: (0, 0, i, 0)),
                  pl.BlockSpec((blk, 3 * H), lambda i: (i, 0))],
        out_specs=pl.BlockSpec((blk, 3 * H), lambda i: (i, 0)),
        out_shape=jax.ShapeDtypeStruct((N, 3 * H), jnp.float32),
    )(pv, av384)


def _pad_idx(idx, total, pad_value):
    pad = total - idx.shape[0]
    p = jnp.concatenate([idx.astype(jnp.int32),
                         jnp.full((pad,), pad_value, jnp.int32)])
    return p.reshape(total // 128, 128)


def _gather_rows(table, idx2d, total):
    return table[idx2d.reshape(-1)]


def _scatter_partials(vals_idx_list, res, nrows):
    acc = res
    for vals, idx2d, nvalid in vals_idx_list:
        acc = acc + jax.ops.segment_sum(
            vals[:nvalid], idx2d.reshape(-1)[:nvalid], num_segments=nrows)
    p1 = res
    return jnp.stack([acc, p1])


def kernel(a_x, a_vec, m_x, a2a_edge_index, a2m_edge_index, m2a_edge_index,
           a2a_edge_weights, a2m_edge_weights, m2a_edge_weights,
           a2a_edge_attr, a2m_edge_attr, m2a_edge_attr, a2a_edge_vecs,
           W_s1, W_s2, W_e, W_a2m, W_m2a, Wq, Wk, Wv, Wo,
           ln_s_g, ln_s_b, ln_f_g, ln_f_b, ln_l_g, ln_l_b):
    a_xn = _ln(a_x, ln_s_g, ln_s_b)
    en = _ln(a2a_edge_attr, ln_f_g, ln_f_b, scale=1.0 / H)
    m_xn = _ln(m_x, ln_l_g, ln_l_b)

    src2 = _pad_idx(a2a_edge_index[0], E_PAD, 0)
    dst2 = _pad_idx(a2a_edge_index[1], E_PAD, N)
    asrc2 = _pad_idx(a2m_edge_index[0], EAM_PAD, 0)
    adst2 = _pad_idx(a2m_edge_index[1], EAM_PAD, M)
    msrc2 = _pad_idx(m2a_edge_index[0], EAM_PAD, 0)
    mdst2 = _pad_idx(m2a_edge_index[1], EAM_PAD, N)
    a_vecf = a_vec.reshape(N, 3 * H)
    a_vecT = jnp.transpose(a_vec, (1, 0, 2))

    gs = _gather_rows(a_xn, src2, E_PAD)
    gd = _gather_rows(a_xn, dst2, E_PAD)
    gv = _gather_rows(a_vecf, src2, E_PAD)
    ga = _gather_rows(a_xn, asrc2, EAM_PAD)
    gm = _gather_rows(m_xn, msrc2, EAM_PAD)

    w2d = a2a_edge_weights.reshape(E, 1)
    v1, s_e_out, v20, v21, v22 = _edge_math(
        gs, gd, gv, en, w2d, a2a_edge_vecs, a2a_edge_attr, W_s1, W_s2, W_e)
    va2m = _pair_math(ga, a2m_edge_attr, a2m_edge_weights.reshape(EAM, 1),
                      W_a2m, EAM, EAM_PAD)
    vm2a = _pair_math(gm, m2a_edge_attr, m2a_edge_weights.reshape(EAM, 1),
                      W_m2a, EAM, EAM_PAD)

    l_m = _mha(m_xn, Wq, Wk, Wv, Wo, m_x)

    pa = _scatter_partials([(v1, dst2, E), (vm2a, mdst2, EAM)], a_x, N)
    pm = _scatter_partials([(va2m, adst2, EAM)], m_x, M)
    pvs = []
    for c, vc in enumerate((v20, v21, v22)):
        pvs.append(_scatter_partials([(vc, dst2, E)], a_vecT[c], N))
    pv = jnp.stack(pvs)

    out_a_x = _combine(pa, a_x, -1.0)
    out_m_x = _combine(pm, l_m, 1.0)
    out_a_vec = _combine_vec(pv, a_vecf).reshape(N, 3, H)
    return out_a_x, out_m_x, out_a_vec, s_e_out


# speedup vs baseline: 7.4874x; 1.6876x over previous
import functools

import jax
import jax.numpy as jnp
from jax import lax
from jax.experimental import pallas as pl
from jax.experimental.pallas import tpu as pltpu
from jax.experimental.pallas import tpu_sc as plsc

H = 128
N = 10000
M = 8192
NG = 512
NH = 8
E = 160000
EAM = 80000
E_PAD = 163840
EAM_PAD = 81920
N_ACC = N + 16


def _ln_body(x_ref, g_ref, b_ref, o_ref, *, scale):
    x = x_ref[...]
    mu = jnp.mean(x, axis=-1, keepdims=True)
    var = jnp.mean((x - mu) ** 2, axis=-1, keepdims=True)
    o_ref[...] = ((x - mu) / jnp.sqrt(var + 1e-5) * g_ref[...] + b_ref[...]) * scale


def _ln(x, g, b, scale=1.0, blk=512):
    n = x.shape[0]
    grid = (n + blk - 1) // blk
    return pl.pallas_call(
        functools.partial(_ln_body, scale=scale),
        grid=(grid,),
        in_specs=[pl.BlockSpec((blk, H), lambda i: (i, 0)),
                  pl.BlockSpec((1, H), lambda i: (0, 0)),
                  pl.BlockSpec((1, H), lambda i: (0, 0))],
        out_specs=pl.BlockSpec((blk, H), lambda i: (i, 0)),
        out_shape=jax.ShapeDtypeStruct((n, H), jnp.float32),
    )(x, g.reshape(1, H), b.reshape(1, H))


def _edge_body(gs_ref, gd_ref, gv_ref, en_ref, w_ref, ev_ref, ea_ref,
               w1_ref, w2_ref, we_ref,
               v1_ref, se_ref, v20_ref, v21_ref, v22_ref):
    en = en_ref[...]
    w = w_ref[...]
    msg = gs_ref[...] * en * w
    v1_ref[...] = jnp.dot(msg, w1_ref[...], preferred_element_type=jnp.float32)
    gate = jnp.dot(msg, w2_ref[...], preferred_element_type=jnp.float32)
    se = jnp.dot(gs_ref[...] * gd_ref[...], we_ref[...],
                 preferred_element_type=jnp.float32)
    se_ref[...] = se + ea_ref[...]
    for c, vo in enumerate((v20_ref, v21_ref, v22_ref)):
        vo[...] = (gv_ref[:, c * H:(c + 1) * H] * en
                   + ev_ref[:, c:c + 1] * gate) * w


def _edge_math(gs, gd, gv, en, w2d, ev, ea, W_s1, W_s2, W_e, blk=1024):
    grid = pl.cdiv(E, blk)
    outs = pl.pallas_call(
        _edge_body,
        grid=(grid,),
        in_specs=[pl.BlockSpec((blk, H), lambda i: (i, 0)),
                  pl.BlockSpec((blk, H), lambda i: (i, 0)),
                  pl.BlockSpec((blk, 3 * H), lambda i: (i, 0)),
                  pl.BlockSpec((blk, H), lambda i: (i, 0)),
                  pl.BlockSpec((blk, 1), lambda i: (i, 0)),
                  pl.BlockSpec((blk, 3), lambda i: (i, 0)),
                  pl.BlockSpec((blk, H), lambda i: (i, 0)),
                  pl.BlockSpec((H, H), lambda i: (0, 0)),
                  pl.BlockSpec((H, H), lambda i: (0, 0)),
                  pl.BlockSpec((H, H), lambda i: (0, 0))],
        out_specs=[pl.BlockSpec((blk, H), lambda i: (i, 0)),
                   pl.BlockSpec((blk, H), lambda i: (i, 0)),
                   pl.BlockSpec((blk, H), lambda i: (i, 0)),
                   pl.BlockSpec((blk, H), lambda i: (i, 0)),
                   pl.BlockSpec((blk, H), lambda i: (i, 0))],
        out_shape=[jax.ShapeDtypeStruct((E_PAD, H), jnp.float32),
                   jax.ShapeDtypeStruct((E, H), jnp.float32),
                   jax.ShapeDtypeStruct((E_PAD, H), jnp.float32),
                   jax.ShapeDtypeStruct((E_PAD, H), jnp.float32),
                   jax.ShapeDtypeStruct((E_PAD, H), jnp.float32)],
    )(gs, gd, gv, en, w2d, ev, ea, W_s1, W_s2, W_e)
    return outs


def _pair_body(g_ref, attr_ref, w_ref, wm_ref, o_ref):
    o_ref[...] = jnp.dot(g_ref[...] * attr_ref[...] * w_ref[...], wm_ref[...],
                         preferred_element_type=jnp.float32)


def _pair_math(g, attr, w2d, Wm, nedge, npad, blk=1024):
    grid = pl.cdiv(nedge, blk)
    return pl.pallas_call(
        _pair_body,
        grid=(grid,),
        in_specs=[pl.BlockSpec((blk, H), lambda i: (i, 0)),
                  pl.BlockSpec((blk, H), lambda i: (i, 0)),
                  pl.BlockSpec((blk, 1), lambda i: (i, 0)),
                  pl.BlockSpec((H, H), lambda i: (0, 0))],
        out_specs=pl.BlockSpec((blk, H), lambda i: (i, 0)),
        out_shape=jax.ShapeDtypeStruct((npad, H), jnp.float32),
    )(g, attr, w2d, Wm)


def _mha_body(x_ref, wq_ref, wk_ref, wv_ref, wo_ref, mx_ref, o_ref):
    x = x_ref[...]
    q = jnp.dot(x, wq_ref[...], preferred_element_type=jnp.float32)
    k = jnp.dot(x, wk_ref[...], preferred_element_type=jnp.float32)
    v = jnp.dot(x, wv_ref[...], preferred_element_type=jnp.float32)
    hd = H // NH
    outs = []
    for h in range(NH):
        qh = q[:, h * hd:(h + 1) * hd]
        kh = k[:, h * hd:(h + 1) * hd]
        vh = v[:, h * hd:(h + 1) * hd]
        att = lax.dot_general(qh, kh, (((1,), (1,)), ((), ())),
                              preferred_element_type=jnp.float32) * (1.0 / 4.0)
        att = jax.nn.softmax(att, axis=-1)
        outs.append(jnp.dot(att, vh, preferred_element_type=jnp.float32))
    o = jnp.concatenate(outs, axis=1)
    o_ref[...] = (jnp.dot(o, wo_ref[...], preferred_element_type=jnp.float32)
                  - mx_ref[...])


def _mha(m_xn, Wq, Wk, Wv, Wo, m_x):
    grid = M // NG
    return pl.pallas_call(
        _mha_body,
        grid=(grid,),
        in_specs=[pl.BlockSpec((NG, H), lambda i: (i, 0)),
                  pl.BlockSpec((H, H), lambda i: (0, 0)),
                  pl.BlockSpec((H, H), lambda i: (0, 0)),
                  pl.BlockSpec((H, H), lambda i: (0, 0)),
                  pl.BlockSpec((H, H), lambda i: (0, 0)),
                  pl.BlockSpec((NG, H), lambda i: (i, 0))],
        out_specs=pl.BlockSpec((NG, H), lambda i: (i, 0)),
        out_shape=jax.ShapeDtypeStruct((M, H), jnp.float32),
    )(m_xn, Wq, Wk, Wv, Wo, m_x)


def _comb_body(p_ref, r_ref, o_ref, *, sign):
    o_ref[...] = p_ref[0] + p_ref[1] + sign * r_ref[...]


def _combine(parts, res, sign, blk=512):
    n, d = res.shape
    grid = (n + blk - 1) // blk
    return pl.pallas_call(
        functools.partial(_comb_body, sign=sign),
        grid=(grid,),
        in_specs=[pl.BlockSpec((2, blk, d), lambda i: (0, i, 0)),
                  pl.BlockSpec((blk, d), lambda i: (i, 0))],
        out_specs=pl.BlockSpec((blk, d), lambda i: (i, 0)),
        out_shape=jax.ShapeDtypeStruct((n, d), jnp.float32),
    )(parts, res)


def _combv_body(p_ref, r_ref, o_ref):
    for c in range(3):
        o_ref[:, c * H:(c + 1) * H] = (p_ref[c, 0] + p_ref[c, 1]
                                       - r_ref[:, c * H:(c + 1) * H])


def _combine_vec(pv, av384, blk=512):
    grid = (N + blk - 1) // blk
    return pl.pallas_call(
        _combv_body,
        grid=(grid,),
        in_specs=[pl.BlockSpec((3, 2, blk, H), lambda i: (0, 0, i, 0)),
                  pl.BlockSpec((blk, 3 * H), lambda i: (i, 0))],
        out_specs=pl.BlockSpec((blk, 3 * H), lambda i: (i, 0)),
        out_shape=jax.ShapeDtypeStruct((N, 3 * H), jnp.float32),
    )(pv, av384)


def _pad_idx(idx, total, pad_value):
    pad = total - idx.shape[0]
    p = jnp.concatenate([idx.astype(jnp.int32),
                         jnp.full((pad,), pad_value, jnp.int32)])
    return p.reshape(total // 128, 128)


_SC_MESH = plsc.VectorSubcoreMesh(core_axis_name="c", subcore_axis_name="s",
                                  num_cores=2, num_subcores=16)
_NW = 32


def _sc_gather_all(a_xn, a_vecf, m_xn, src2, dst2g, asrc2, msrc2):
    @functools.partial(
        pl.kernel,
        out_type=[jax.ShapeDtypeStruct((E_PAD, H), jnp.float32),
                  jax.ShapeDtypeStruct((E_PAD, H), jnp.float32),
                  jax.ShapeDtypeStruct((E_PAD, 3 * H), jnp.float32),
                  jax.ShapeDtypeStruct((EAM_PAD, H), jnp.float32),
                  jax.ShapeDtypeStruct((EAM_PAD, H), jnp.float32)],
        mesh=_SC_MESH,
        scratch_types=[pltpu.VMEM((128,), jnp.int32),
                       pltpu.VMEM((128, H), jnp.float32),
                       pltpu.VMEM((128, 3 * H), jnp.float32),
                       pltpu.SemaphoreType.DMA],
    )
    def k(axn_hbm, avec_hbm, mxn_hbm, src_hbm, dst_hbm, asrc_hbm, msrc_hbm,
          gs_hbm, gd_hbm, gv_hbm, ga_hbm, gm_hbm, idx_v, row_v, vrow_v, sem):
        wid = lax.axis_index("c") * 16 + lax.axis_index("s")

        def gather_one(idx2d, table, out, rpw, rowbuf):
            def body(j, carry):
                r = wid * rpw + j
                pltpu.sync_copy(idx2d.at[r], idx_v)
                pltpu.async_copy(table.at[idx_v], rowbuf, sem).wait()
                pltpu.sync_copy(rowbuf, out.at[pl.ds(r * 128, 128)])
                return carry
            lax.fori_loop(0, rpw, body, 0)

        gather_one(src_hbm, axn_hbm, gs_hbm, E_PAD // 128 // _NW, row_v)
        gather_one(dst_hbm, axn_hbm, gd_hbm, E_PAD // 128 // _NW, row_v)
        gather_one(src_hbm, avec_hbm, gv_hbm, E_PAD // 128 // _NW, vrow_v)
        gather_one(asrc_hbm, axn_hbm, ga_hbm, EAM_PAD // 128 // _NW, row_v)
        gather_one(msrc_hbm, mxn_hbm, gm_hbm, EAM_PAD // 128 // _NW, row_v)

    return k(a_xn, a_vecf, m_xn, src2, dst2g, asrc2, msrc2)


def _sc_scatter_all(v1, vm2a, va2m, v20, v21, v22,
                    dst2, mdst2, adst2, a_x, m_x, a_vecT):
    @functools.partial(
        pl.kernel,
        out_type=[jax.ShapeDtypeStruct((2, N, H), jnp.float32),
                  jax.ShapeDtypeStruct((2, M, H), jnp.float32),
                  jax.ShapeDtypeStruct((3, 2, N, H), jnp.float32)],
        mesh=_SC_MESH,
        scratch_types=[pltpu.VMEM((128,), jnp.int32),
                       pltpu.VMEM((128, H), jnp.float32),
                       pltpu.VMEM((128, H), jnp.float32),
                       pltpu.VMEM_SHARED((N_ACC, H), jnp.float32)],
    )
    def k(v1_hbm, vm2a_hbm, va2m_hbm, v20_hbm, v21_hbm, v22_hbm,
          dst_hbm, mdst_hbm, adst_hbm, ax_hbm, mx_hbm, avT_hbm,
          pa_hbm, pm_hbm, pv_hbm, idx_v, val_v, buf_v, shared):
        cid = lax.axis_index("c")
        sid = lax.axis_index("s")

        def accum(vals, idx2d, nrows_idx):
            half = nrows_idx // 2
            rpt = half // 16
            def body(j, carry):
                r = cid * half + sid * rpt + j
                pltpu.sync_copy(idx2d.at[r], idx_v)
                pltpu.sync_copy(vals.at[pl.ds(r * 128, 128)], val_v)
                pltpu.sync_copy(val_v, shared.at[idx_v], add=True)
                return carry
            lax.fori_loop(0, rpt, body, 0)

        def sweep(nrows, do_chunk):
            nfull = nrows // 128
            tail = nrows - nfull * 128
            for r in range((nfull + 15) // 16):
                c_id = sid + r * 16
                @pl.when(c_id < nfull)
                def _():
                    do_chunk(pl.multiple_of(c_id * 128, 128), 128)
            if tail:
                @pl.when(sid == 15)
                def _():
                    do_chunk(nfull * 128, tail)

        def phase(init_src, streams, out_dst, nrows):
            def init_chunk(s, ch):
                pltpu.sync_copy(init_src(s, ch), buf_v.at[pl.ds(0, ch)])
                pltpu.sync_copy(buf_v.at[pl.ds(0, ch)], shared.at[pl.ds(s, ch)])
            sweep(nrows, init_chunk)
            plsc.subcore_barrier()
            for vals, idx2d, nri in streams:
                accum(vals, idx2d, nri)
            plsc.subcore_barrier()
            def out_chunk(s, ch):
                pltpu.sync_copy(shared.at[pl.ds(s, ch)], buf_v.at[pl.ds(0, ch)])
                pltpu.sync_copy(buf_v.at[pl.ds(0, ch)], out_dst(s, ch))
            sweep(nrows, out_chunk)
            plsc.subcore_barrier()

        phase(lambda s, ch: ax_hbm.at[pl.ds(s, ch)],
              [(v1_hbm, dst_hbm, E_PAD // 128),
               (vm2a_hbm, mdst_hbm, EAM_PAD // 128)],
              lambda s, ch: pa_hbm.at[cid, pl.ds(s, ch)], N)
        phase(lambda s, ch: mx_hbm.at[pl.ds(s, ch)],
              [(va2m_hbm, adst_hbm, EAM_PAD // 128)],
              lambda s, ch: pm_hbm.at[cid, pl.ds(s, ch)], M)
        for c, vc in enumerate((v20_hbm, v21_hbm, v22_hbm)):
            phase(lambda s, ch: avT_hbm.at[c, pl.ds(s, ch)],
                  [(vc, dst_hbm, E_PAD // 128)],
                  lambda s, ch: pv_hbm.at[c, cid, pl.ds(s, ch)], N)

    return k(v1, vm2a, va2m, v20, v21, v22, dst2, mdst2, adst2,
             a_x, m_x, a_vecT)


def kernel(a_x, a_vec, m_x, a2a_edge_index, a2m_edge_index, m2a_edge_index,
           a2a_edge_weights, a2m_edge_weights, m2a_edge_weights,
           a2a_edge_attr, a2m_edge_attr, m2a_edge_attr, a2a_edge_vecs,
           W_s1, W_s2, W_e, W_a2m, W_m2a, Wq, Wk, Wv, Wo,
           ln_s_g, ln_s_b, ln_f_g, ln_f_b, ln_l_g, ln_l_b):
    a_xn = _ln(a_x, ln_s_g, ln_s_b)
    en = _ln(a2a_edge_attr, ln_f_g, ln_f_b, scale=1.0 / H)
    m_xn = _ln(m_x, ln_l_g, ln_l_b)

    src2 = _pad_idx(a2a_edge_index[0], E_PAD, 0)
    dst2 = _pad_idx(a2a_edge_index[1], E_PAD, N)
    dst2g = _pad_idx(a2a_edge_index[1], E_PAD, 0)
    asrc2 = _pad_idx(a2m_edge_index[0], EAM_PAD, 0)
    adst2 = _pad_idx(a2m_edge_index[1], EAM_PAD, M)
    msrc2 = _pad_idx(m2a_edge_index[0], EAM_PAD, 0)
    mdst2 = _pad_idx(m2a_edge_index[1], EAM_PAD, N)
    a_vecf = a_vec.reshape(N, 3 * H)
    a_vecT = jnp.transpose(a_vec, (1, 0, 2))

    gs, gd, gv, ga, gm = _sc_gather_all(a_xn, a_vecf, m_xn,
                                        src2, dst2g, asrc2, msrc2)

    w2d = a2a_edge_weights.reshape(E, 1)
    v1, s_e_out, v20, v21, v22 = _edge_math(
        gs, gd, gv, en, w2d, a2a_edge_vecs, a2a_edge_attr, W_s1, W_s2, W_e)
    va2m = _pair_math(ga, a2m_edge_attr, a2m_edge_weights.reshape(EAM, 1),
                      W_a2m, EAM, EAM_PAD)
    vm2a = _pair_math(gm, m2a_edge_attr, m2a_edge_weights.reshape(EAM, 1),
                      W_m2a, EAM, EAM_PAD)

    l_m = _mha(m_xn, Wq, Wk, Wv, Wo, m_x)

    pa, pm, pv = _sc_scatter_all(v1, vm2a, va2m, v20, v21, v22,
                                 dst2, mdst2, adst2, a_x, m_x, a_vecT)

    out_a_x = _combine(pa, a_x, -1.0)
    out_m_x = _combine(pm, l_m, 1.0)
    out_a_vec = _combine_vec(pv, a_vecf).reshape(N, 3, H)
    return out_a_x, out_m_x, out_a_vec, s_e_out


# speedup vs baseline: 7.8466x; 1.0480x over previous
import functools

import jax
import jax.numpy as jnp
from jax import lax
from jax.experimental import pallas as pl
from jax.experimental.pallas import tpu as pltpu
from jax.experimental.pallas import tpu_sc as plsc

H = 128
N = 10000
M = 8192
NG = 512
NH = 8
E = 160000
EAM = 80000
E_PAD = 163840
EAM_PAD = 81920
N_ACC = N + 16


def _ln_body(x_ref, g_ref, b_ref, o_ref, *, scale):
    x = x_ref[...]
    mu = jnp.mean(x, axis=-1, keepdims=True)
    var = jnp.mean((x - mu) ** 2, axis=-1, keepdims=True)
    o_ref[...] = ((x - mu) / jnp.sqrt(var + 1e-5) * g_ref[...] + b_ref[...]) * scale


def _ln(x, g, b, scale=1.0, blk=512):
    n = x.shape[0]
    grid = (n + blk - 1) // blk
    return pl.pallas_call(
        functools.partial(_ln_body, scale=scale),
        grid=(grid,),
        in_specs=[pl.BlockSpec((blk, H), lambda i: (i, 0)),
                  pl.BlockSpec((1, H), lambda i: (0, 0)),
                  pl.BlockSpec((1, H), lambda i: (0, 0))],
        out_specs=pl.BlockSpec((blk, H), lambda i: (i, 0)),
        out_shape=jax.ShapeDtypeStruct((n, H), jnp.float32),
    )(x, g.reshape(1, H), b.reshape(1, H))


def _edge_body(gs_ref, gd_ref, gv0_ref, gv1_ref, gv2_ref,
               en_ref, w_ref, ev_ref, ea_ref,
               w1_ref, w2_ref, we_ref,
               v1_ref, se_ref, v20_ref, v21_ref, v22_ref):
    en = en_ref[...]
    w = w_ref[...]
    msg = gs_ref[...] * en * w
    v1_ref[...] = jnp.dot(msg, w1_ref[...], preferred_element_type=jnp.float32)
    gate = jnp.dot(msg, w2_ref[...], preferred_element_type=jnp.float32)
    se = jnp.dot(gs_ref[...] * gd_ref[...], we_ref[...],
                 preferred_element_type=jnp.float32)
    se_ref[...] = se + ea_ref[...]
    for c, (gv, vo) in enumerate(((gv0_ref, v20_ref), (gv1_ref, v21_ref),
                                  (gv2_ref, v22_ref))):
        vo[...] = (gv[...] * en + ev_ref[:, c:c + 1] * gate) * w


def _edge_math(gs, gd, gv0, gv1, gv2, en, w2d, ev, ea, W_s1, W_s2, W_e,
               blk=1024):
    grid = pl.cdiv(E, blk)
    outs = pl.pallas_call(
        _edge_body,
        grid=(grid,),
        in_specs=[pl.BlockSpec((blk, H), lambda i: (i, 0)),
                  pl.BlockSpec((blk, H), lambda i: (i, 0)),
                  pl.BlockSpec((blk, H), lambda i: (i, 0)),
                  pl.BlockSpec((blk, H), lambda i: (i, 0)),
                  pl.BlockSpec((blk, H), lambda i: (i, 0)),
                  pl.BlockSpec((blk, H), lambda i: (i, 0)),
                  pl.BlockSpec((blk, 1), lambda i: (i, 0)),
                  pl.BlockSpec((blk, 3), lambda i: (i, 0)),
                  pl.BlockSpec((blk, H), lambda i: (i, 0)),
                  pl.BlockSpec((H, H), lambda i: (0, 0)),
                  pl.BlockSpec((H, H), lambda i: (0, 0)),
                  pl.BlockSpec((H, H), lambda i: (0, 0))],
        out_specs=[pl.BlockSpec((blk, H), lambda i: (i, 0)),
                   pl.BlockSpec((blk, H), lambda i: (i, 0)),
                   pl.BlockSpec((blk, H), lambda i: (i, 0)),
                   pl.BlockSpec((blk, H), lambda i: (i, 0)),
                   pl.BlockSpec((blk, H), lambda i: (i, 0))],
        out_shape=[jax.ShapeDtypeStruct((E_PAD, H), jnp.float32),
                   jax.ShapeDtypeStruct((E, H), jnp.float32),
                   jax.ShapeDtypeStruct((E_PAD, H), jnp.float32),
                   jax.ShapeDtypeStruct((E_PAD, H), jnp.float32),
                   jax.ShapeDtypeStruct((E_PAD, H), jnp.float32)],
    )(gs, gd, gv0, gv1, gv2, en, w2d, ev, ea, W_s1, W_s2, W_e)
    return outs


def _pair_body(g_ref, attr_ref, w_ref, wm_ref, o_ref):
    o_ref[...] = jnp.dot(g_ref[...] * attr_ref[...] * w_ref[...], wm_ref[...],
                         preferred_element_type=jnp.float32)


def _pair_math(g, attr, w2d, Wm, nedge, npad, blk=1024):
    grid = pl.cdiv(nedge, blk)
    return pl.pallas_call(
        _pair_body,
        grid=(grid,),
        in_specs=[pl.BlockSpec((blk, H), lambda i: (i, 0)),
                  pl.BlockSpec((blk, H), lambda i: (i, 0)),
                  pl.BlockSpec((blk, 1), lambda i: (i, 0)),
                  pl.BlockSpec((H, H), lambda i: (0, 0))],
        out_specs=pl.BlockSpec((blk, H), lambda i: (i, 0)),
        out_shape=jax.ShapeDtypeStruct((npad, H), jnp.float32),
    )(g, attr, w2d, Wm)


def _mha_body(x_ref, wq_ref, wk_ref, wv_ref, wo_ref, mx_ref, o_ref):
    x = x_ref[...]
    q = jnp.dot(x, wq_ref[...], preferred_element_type=jnp.float32)
    k = jnp.dot(x, wk_ref[...], preferred_element_type=jnp.float32)
    v = jnp.dot(x, wv_ref[...], preferred_element_type=jnp.float32)
    hd = H // NH
    outs = []
    for h in range(NH):
        qh = q[:, h * hd:(h + 1) * hd]
        kh = k[:, h * hd:(h + 1) * hd]
        vh = v[:, h * hd:(h + 1) * hd]
        att = lax.dot_general(qh, kh, (((1,), (1,)), ((), ())),
                              preferred_element_type=jnp.float32) * (1.0 / 4.0)
        att = jax.nn.softmax(att, axis=-1)
        outs.append(jnp.dot(att, vh, preferred_element_type=jnp.float32))
    o = jnp.concatenate(outs, axis=1)
    o_ref[...] = (jnp.dot(o, wo_ref[...], preferred_element_type=jnp.float32)
                  + mx_ref[...])


def _mha(m_xn, Wq, Wk, Wv, Wo, m_x):
    grid = M // NG
    return pl.pallas_call(
        _mha_body,
        grid=(grid,),
        in_specs=[pl.BlockSpec((NG, H), lambda i: (i, 0)),
                  pl.BlockSpec((H, H), lambda i: (0, 0)),
                  pl.BlockSpec((H, H), lambda i: (0, 0)),
                  pl.BlockSpec((H, H), lambda i: (0, 0)),
                  pl.BlockSpec((H, H), lambda i: (0, 0)),
                  pl.BlockSpec((NG, H), lambda i: (i, 0))],
        out_specs=pl.BlockSpec((NG, H), lambda i: (i, 0)),
        out_shape=jax.ShapeDtypeStruct((M, H), jnp.float32),
    )(m_xn, Wq, Wk, Wv, Wo, m_x)


def _comb_body(p_ref, r_ref, o_ref, *, sign):
    o_ref[...] = p_ref[0] + p_ref[1] + sign * r_ref[...]


def _combine(parts, res, sign, blk=512):
    n, d = res.shape
    grid = (n + blk - 1) // blk
    return pl.pallas_call(
        functools.partial(_comb_body, sign=sign),
        grid=(grid,),
        in_specs=[pl.BlockSpec((2, blk, d), lambda i: (0, i, 0)),
                  pl.BlockSpec((blk, d), lambda i: (i, 0))],
        out_specs=pl.BlockSpec((blk, d), lambda i: (i, 0)),
        out_shape=jax.ShapeDtypeStruct((n, d), jnp.float32),
    )(parts, res)


def _combv_body(p_ref, r_ref, o_ref):
    for c in range(3):
        o_ref[:, c * H:(c + 1) * H] = (p_ref[c, 0] + p_ref[c, 1]
                                       - r_ref[:, c * H:(c + 1) * H])


def _combine_vec(pv, av384, blk=512):
    grid = (N + blk - 1) // blk
    return pl.pallas_call(
        _combv_body,
        grid=(grid,),
        in_specs=[pl.BlockSpec((3, 2, blk, H), lambda i: (0, 0, i, 0)),
                  pl.BlockSpec((blk, 3 * H), lambda i: (i, 0))],
        out_specs=pl.BlockSpec((blk, 3 * H), lambda i: (i, 0)),
        out_shape=jax.ShapeDtypeStruct((N, 3 * H), jnp.float32),
    )(pv, av384)


def _pad_idx(idx, total, pad_value):
    pad = total - idx.shape[0]
    p = jnp.concatenate([idx.astype(jnp.int32),
                         jnp.full((pad,), pad_value, jnp.int32)])
    return p.reshape(total // 128, 128)


_NW = 32


@functools.cache
def _sc_mesh():
    return plsc.VectorSubcoreMesh(core_axis_name="c", subcore_axis_name="s",
                                  num_cores=2, num_subcores=16)


def _sc_gather_all(a_xn, avF, m_xn, src3, src3_1, src3_2, dst3, asrc3, msrc3):
    @functools.partial(
        pl.kernel,
        out_type=[jax.ShapeDtypeStruct((E_PAD, H), jnp.float32),
                  jax.ShapeDtypeStruct((E_PAD, H), jnp.float32),
                  jax.ShapeDtypeStruct((E_PAD, H), jnp.float32),
                  jax.ShapeDtypeStruct((E_PAD, H), jnp.float32),
                  jax.ShapeDtypeStruct((E_PAD, H), jnp.float32),
                  jax.ShapeDtypeStruct((EAM_PAD, H), jnp.float32),
                  jax.ShapeDtypeStruct((EAM_PAD, H), jnp.float32)],
        mesh=_sc_mesh(),
        scratch_types=[pltpu.VMEM((E_PAD // 128 // _NW, 128), jnp.int32),
                       pltpu.VMEM((128, H), jnp.float32),
                       pltpu.VMEM((128, H), jnp.float32),
                       pltpu.SemaphoreType.DMA,
                       pltpu.SemaphoreType.DMA],
    )
    def k(axn_hbm, avf_hbm, mxn_hbm,
          src_hbm, src1_hbm, src2_hbm, dst_hbm, asrc_hbm, msrc_hbm,
          gs_hbm, gd_hbm, gv0_hbm, gv1_hbm, gv2_hbm, ga_hbm, gm_hbm,
          idx_all, buf_a, buf_b, sem_a, sem_b):
        wid = lax.axis_index("c") * 16 + lax.axis_index("s")

        def gather_one(idx3, table, out, rpw):
            pltpu.sync_copy(idx3.at[wid], idx_all.at[pl.ds(0, rpw)])

            def body(j, carry):
                c0 = 2 * j
                r0 = wid * rpw + c0
                da = pltpu.async_copy(table.at[idx_all.at[c0]], buf_a, sem_a)
                db = pltpu.async_copy(table.at[idx_all.at[c0 + 1]], buf_b,
                                      sem_b)
                da.wait()
                pltpu.sync_copy(buf_a, out.at[pl.ds(r0 * 128, 128)])
                db.wait()
                pltpu.sync_copy(buf_b, out.at[pl.ds((r0 + 1) * 128, 128)])
                return carry
            lax.fori_loop(0, rpw // 2, body, 0)

        rpw_e = E_PAD // 128 // _NW
        rpw_am = EAM_PAD // 128 // _NW
        gather_one(src_hbm, axn_hbm, gs_hbm, rpw_e)
        gather_one(dst_hbm, axn_hbm, gd_hbm, rpw_e)
        gather_one(src_hbm, avf_hbm, gv0_hbm, rpw_e)
        gather_one(src1_hbm, avf_hbm, gv1_hbm, rpw_e)
        gather_one(src2_hbm, avf_hbm, gv2_hbm, rpw_e)
        gather_one(asrc_hbm, axn_hbm, ga_hbm, rpw_am)
        gather_one(msrc_hbm, mxn_hbm, gm_hbm, rpw_am)

    return k(a_xn, avF, m_xn, src3, src3_1, src3_2, dst3, asrc3, msrc3)


def _sc_scatter_all(v1, vm2a, va2m, v20, v21, v22,
                    dst3s, mdst3s, adst3s, a_x, l_m, avT):
    @functools.partial(
        pl.kernel,
        out_type=[jax.ShapeDtypeStruct((N, H), jnp.float32),
                  jax.ShapeDtypeStruct((M, H), jnp.float32),
                  jax.ShapeDtypeStruct((3, N, H), jnp.float32)],
        mesh=_sc_mesh(),
        scratch_types=[pltpu.VMEM((E_PAD // 128 // 16, 128), jnp.int32),
                       pltpu.VMEM((128, H), jnp.float32),
                       pltpu.VMEM((128, H), jnp.float32),
                       pltpu.VMEM_SHARED((N_ACC, H), jnp.float32),
                       pltpu.SemaphoreType.DMA,
                       pltpu.SemaphoreType.DMA],
    )
    def k(v1_hbm, vm2a_hbm, va2m_hbm, v20_hbm, v21_hbm, v22_hbm,
          dst_hbm, mdst_hbm, adst_hbm, ax_hbm, lm_hbm, avT_hbm,
          oax_hbm, omx_hbm, ovT_hbm,
          idx_all, val_a, val_b, shared, sem_a, sem_b):
        cid = lax.axis_index("c")
        sid = lax.axis_index("s")

        def sweep(nrows, do_chunk):
            nfull = nrows // 128
            tail = nrows - nfull * 128
            for r in range((nfull + 15) // 16):
                c_id = sid + r * 16
                @pl.when(c_id < nfull)
                def _():
                    do_chunk(pl.multiple_of(c_id * 128, 128), 128)
            if tail:
                @pl.when(sid == 15)
                def _():
                    do_chunk(nfull * 128, tail)

        def accum(vals, idx3, rpt):
            pltpu.sync_copy(idx3.at[sid], idx_all.at[pl.ds(0, rpt)])

            def body(j, carry):
                c0 = 2 * j
                r0 = sid * rpt + c0
                da = pltpu.async_copy(vals.at[pl.ds(r0 * 128, 128)],
                                      val_a, sem_a)
                db = pltpu.async_copy(vals.at[pl.ds((r0 + 1) * 128, 128)],
                                      val_b, sem_b)
                da.wait()
                pltpu.sync_copy(val_a, shared.at[idx_all.at[c0]], add=True)
                db.wait()
                pltpu.sync_copy(val_b, shared.at[idx_all.at[c0 + 1]],
                                add=True)
                return carry
            lax.fori_loop(0, rpt // 2, body, 0)

        def phase(init_src, streams, out_dst, nrows):
            def init_chunk(s, ch):
                pltpu.sync_copy(init_src(s, ch), val_a.at[pl.ds(0, ch)])
                pltpu.sync_copy(val_a.at[pl.ds(0, ch)], shared.at[pl.ds(s, ch)])
            sweep(nrows, init_chunk)
            plsc.subcore_barrier()
            for vals, idx3, rpt in streams:
                accum(vals, idx3, rpt)
            plsc.subcore_barrier()
            def out_chunk(s, ch):
                pltpu.sync_copy(shared.at[pl.ds(s, ch)], val_a.at[pl.ds(0, ch)])
                pltpu.sync_copy(val_a.at[pl.ds(0, ch)], out_dst(s, ch))
            sweep(nrows, out_chunk)
            plsc.subcore_barrier()

        rpt_e = E_PAD // 128 // 16
        rpt_am = EAM_PAD // 128 // 16

        @pl.when(cid == 0)
        def _():
            phase(lambda s, ch: ax_hbm.at[pl.ds(s, ch)],
                  [(v1_hbm, dst_hbm, rpt_e), (vm2a_hbm, mdst_hbm, rpt_am)],
                  lambda s, ch: oax_hbm.at[pl.ds(s, ch)], N)
            phase(lambda s, ch: avT_hbm.at[0, pl.ds(s, ch)],
                  [(v20_hbm, dst_hbm, rpt_e)],
                  lambda s, ch: ovT_hbm.at[0, pl.ds(s, ch)], N)

        @pl.when(cid == 1)
        def _():
            phase(lambda s, ch: lm_hbm.at[pl.ds(s, ch)],
                  [(va2m_hbm, adst_hbm, rpt_am)],
                  lambda s, ch: omx_hbm.at[pl.ds(s, ch)], M)
            phase(lambda s, ch: avT_hbm.at[1, pl.ds(s, ch)],
                  [(v21_hbm, dst_hbm, rpt_e)],
                  lambda s, ch: ovT_hbm.at[1, pl.ds(s, ch)], N)
            phase(lambda s, ch: avT_hbm.at[2, pl.ds(s, ch)],
                  [(v22_hbm, dst_hbm, rpt_e)],
                  lambda s, ch: ovT_hbm.at[2, pl.ds(s, ch)], N)

    return k(v1, vm2a, va2m, v20, v21, v22, dst3s, mdst3s, adst3s,
             a_x, l_m, avT)


def kernel(a_x, a_vec, m_x, a2a_edge_index, a2m_edge_index, m2a_edge_index,
           a2a_edge_weights, a2m_edge_weights, m2a_edge_weights,
           a2a_edge_attr, a2m_edge_attr, m2a_edge_attr, a2a_edge_vecs,
           W_s1, W_s2, W_e, W_a2m, W_m2a, Wq, Wk, Wv, Wo,
           ln_s_g, ln_s_b, ln_f_g, ln_f_b, ln_l_g, ln_l_b):
    a_xn = _ln(a_x, ln_s_g, ln_s_b)
    en = _ln(a2a_edge_attr, ln_f_g, ln_f_b, scale=1.0 / H)
    m_xn = _ln(m_x, ln_l_g, ln_l_b)

    src2 = _pad_idx(a2a_edge_index[0], E_PAD, 0)
    dstg2 = _pad_idx(a2a_edge_index[1], E_PAD, 0)
    dsts2 = _pad_idx(a2a_edge_index[1], E_PAD, N)
    asrc2 = _pad_idx(a2m_edge_index[0], EAM_PAD, 0)
    adst2 = _pad_idx(a2m_edge_index[1], EAM_PAD, M)
    msrc2 = _pad_idx(m2a_edge_index[0], EAM_PAD, 0)
    mdst2 = _pad_idx(m2a_edge_index[1], EAM_PAD, N)

    def g32(x):
        return x.reshape(_NW, -1, 128)

    def s16(x):
        return x.reshape(16, -1, 128)

    avT = jnp.transpose(a_vec, (1, 0, 2))
    avF = avT.reshape(3 * N, H)

    gs, gd, gv0, gv1, gv2, ga, gm = _sc_gather_all(
        a_xn, avF, m_xn, g32(src2), g32(src2 + N), g32(src2 + 2 * N),
        g32(dstg2), g32(asrc2), g32(msrc2))

    w2d = a2a_edge_weights.reshape(E, 1)
    v1, s_e_out, v20, v21, v22 = _edge_math(
        gs, gd, gv0, gv1, gv2, en, w2d, a2a_edge_vecs, a2a_edge_attr,
        W_s1, W_s2, W_e)
    va2m = _pair_math(ga, a2m_edge_attr, a2m_edge_weights.reshape(EAM, 1),
                      W_a2m, EAM, EAM_PAD)
    vm2a = _pair_math(gm, m2a_edge_attr, m2a_edge_weights.reshape(EAM, 1),
                      W_m2a, EAM, EAM_PAD)

    l_m = _mha(m_xn, Wq, Wk, Wv, Wo, m_x)

    out_a_x, out_m_x, out_vT = _sc_scatter_all(
        v1, vm2a, va2m, v20, v21, v22,
        s16(dsts2), s16(mdst2), s16(adst2), a_x, l_m, avT)

    out_a_vec = jnp.transpose(out_vT, (1, 0, 2))
    return out_a_x, out_m_x, out_a_vec, s_e_out


# speedup vs baseline: 9.2717x; 1.1816x over previous
import functools

import jax
import jax.numpy as jnp
from jax import lax
from jax.experimental import pallas as pl
from jax.experimental.pallas import tpu as pltpu
from jax.experimental.pallas import tpu_sc as plsc

H = 128
N = 10000
M = 8192
NG = 512
NH = 8
E = 160000
EAM = 80000
E_PAD = 163840
EAM_PAD = 81920
N_ACC = N + 16


def _ln_body(x_ref, g_ref, b_ref, o_ref, *, scale):
    x = x_ref[...]
    mu = jnp.mean(x, axis=-1, keepdims=True)
    var = jnp.mean((x - mu) ** 2, axis=-1, keepdims=True)
    o_ref[...] = ((x - mu) / jnp.sqrt(var + 1e-5) * g_ref[...] + b_ref[...]) * scale


def _ln(x, g, b, scale=1.0, blk=512):
    n = x.shape[0]
    grid = (n + blk - 1) // blk
    return pl.pallas_call(
        functools.partial(_ln_body, scale=scale),
        grid=(grid,),
        in_specs=[pl.BlockSpec((blk, H), lambda i: (i, 0)),
                  pl.BlockSpec((1, H), lambda i: (0, 0)),
                  pl.BlockSpec((1, H), lambda i: (0, 0))],
        out_specs=pl.BlockSpec((blk, H), lambda i: (i, 0)),
        out_shape=jax.ShapeDtypeStruct((n, H), jnp.float32),
    )(x, g.reshape(1, H), b.reshape(1, H))


def _edge_body(gs_ref, gd_ref, gv0_ref, gv1_ref, gv2_ref,
               en_ref, w_ref, ev_ref, ea_ref,
               w1_ref, w2_ref, we_ref,
               v1_ref, se_ref, v20_ref, v21_ref, v22_ref):
    en = en_ref[...]
    w = w_ref[...]
    msg = gs_ref[...] * en * w
    v1_ref[...] = jnp.dot(msg, w1_ref[...], preferred_element_type=jnp.float32)
    gate = jnp.dot(msg, w2_ref[...], preferred_element_type=jnp.float32)
    se = jnp.dot(gs_ref[...] * gd_ref[...], we_ref[...],
                 preferred_element_type=jnp.float32)
    se_ref[...] = se + ea_ref[...]
    for c, (gv, vo) in enumerate(((gv0_ref, v20_ref), (gv1_ref, v21_ref),
                                  (gv2_ref, v22_ref))):
        vo[...] = (gv[...] * en + ev_ref[:, c:c + 1] * gate) * w


def _edge_math(gs, gd, gv0, gv1, gv2, en, w2d, ev, ea, W_s1, W_s2, W_e,
               row0, nloc, blk=1024):
    grid = pl.cdiv(nloc, blk)
    npad = gs.shape[0]
    outs = pl.pallas_call(
        _edge_body,
        grid=(grid,),
        in_specs=[pl.BlockSpec((blk, H), lambda i: (i, 0)),
                  pl.BlockSpec((blk, H), lambda i: (i, 0)),
                  pl.BlockSpec((blk, H), lambda i: (i, 0)),
                  pl.BlockSpec((blk, H), lambda i: (i, 0)),
                  pl.BlockSpec((blk, H), lambda i: (i, 0)),
                  pl.BlockSpec((blk, H), lambda i: (i + row0, 0)),
                  pl.BlockSpec((blk, 1), lambda i: (i + row0, 0)),
                  pl.BlockSpec((blk, 3), lambda i: (i + row0, 0)),
                  pl.BlockSpec((blk, H), lambda i: (i + row0, 0)),
                  pl.BlockSpec((H, H), lambda i: (0, 0)),
                  pl.BlockSpec((H, H), lambda i: (0, 0)),
                  pl.BlockSpec((H, H), lambda i: (0, 0))],
        out_specs=[pl.BlockSpec((blk, H), lambda i: (i, 0)),
                   pl.BlockSpec((blk, H), lambda i: (i, 0)),
                   pl.BlockSpec((blk, H), lambda i: (i, 0)),
                   pl.BlockSpec((blk, H), lambda i: (i, 0)),
                   pl.BlockSpec((blk, H), lambda i: (i, 0))],
        out_shape=[jax.ShapeDtypeStruct((npad, H), jnp.float32),
                   jax.ShapeDtypeStruct((nloc, H), jnp.float32),
                   jax.ShapeDtypeStruct((npad, H), jnp.float32),
                   jax.ShapeDtypeStruct((npad, H), jnp.float32),
                   jax.ShapeDtypeStruct((npad, H), jnp.float32)],
    )(gs, gd, gv0, gv1, gv2, en, w2d, ev, ea, W_s1, W_s2, W_e)
    return outs


def _pair_body(g_ref, attr_ref, w_ref, wm_ref, o_ref):
    o_ref[...] = jnp.dot(g_ref[...] * attr_ref[...] * w_ref[...], wm_ref[...],
                         preferred_element_type=jnp.float32)


def _pair_math(g, attr, w2d, Wm, nedge, npad, blk=1024):
    grid = pl.cdiv(nedge, blk)
    return pl.pallas_call(
        _pair_body,
        grid=(grid,),
        in_specs=[pl.BlockSpec((blk, H), lambda i: (i, 0)),
                  pl.BlockSpec((blk, H), lambda i: (i, 0)),
                  pl.BlockSpec((blk, 1), lambda i: (i, 0)),
                  pl.BlockSpec((H, H), lambda i: (0, 0))],
        out_specs=pl.BlockSpec((blk, H), lambda i: (i, 0)),
        out_shape=jax.ShapeDtypeStruct((npad, H), jnp.float32),
    )(g, attr, w2d, Wm)


def _mha_body(x_ref, wq_ref, wk_ref, wv_ref, wo_ref, mx_ref, o_ref):
    x = x_ref[...]
    q = jnp.dot(x, wq_ref[...], preferred_element_type=jnp.float32)
    k = jnp.dot(x, wk_ref[...], preferred_element_type=jnp.float32)
    v = jnp.dot(x, wv_ref[...], preferred_element_type=jnp.float32)
    hd = H // NH
    outs = []
    for h in range(NH):
        qh = q[:, h * hd:(h + 1) * hd]
        kh = k[:, h * hd:(h + 1) * hd]
        vh = v[:, h * hd:(h + 1) * hd]
        att = lax.dot_general(qh, kh, (((1,), (1,)), ((), ())),
                              preferred_element_type=jnp.float32) * (1.0 / 4.0)
        att = jax.nn.softmax(att, axis=-1)
        outs.append(jnp.dot(att, vh, preferred_element_type=jnp.float32))
    o = jnp.concatenate(outs, axis=1)
    o_ref[...] = (jnp.dot(o, wo_ref[...], preferred_element_type=jnp.float32)
                  + mx_ref[...])


def _mha(m_xn, Wq, Wk, Wv, Wo, m_x):
    grid = M // NG
    return pl.pallas_call(
        _mha_body,
        grid=(grid,),
        in_specs=[pl.BlockSpec((NG, H), lambda i: (i, 0)),
                  pl.BlockSpec((H, H), lambda i: (0, 0)),
                  pl.BlockSpec((H, H), lambda i: (0, 0)),
                  pl.BlockSpec((H, H), lambda i: (0, 0)),
                  pl.BlockSpec((H, H), lambda i: (0, 0)),
                  pl.BlockSpec((NG, H), lambda i: (i, 0))],
        out_specs=pl.BlockSpec((NG, H), lambda i: (i, 0)),
        out_shape=jax.ShapeDtypeStruct((M, H), jnp.float32),
    )(m_xn, Wq, Wk, Wv, Wo, m_x)


def _comb_body(p_ref, r_ref, o_ref, *, sign):
    o_ref[...] = p_ref[0] + p_ref[1] + sign * r_ref[...]


def _combine(parts, res, sign, blk=512):
    n, d = res.shape
    grid = (n + blk - 1) // blk
    return pl.pallas_call(
        functools.partial(_comb_body, sign=sign),
        grid=(grid,),
        in_specs=[pl.BlockSpec((2, blk, d), lambda i: (0, i, 0)),
                  pl.BlockSpec((blk, d), lambda i: (i, 0))],
        out_specs=pl.BlockSpec((blk, d), lambda i: (i, 0)),
        out_shape=jax.ShapeDtypeStruct((n, d), jnp.float32),
    )(parts, res)


def _combv_body(p_ref, r_ref, o_ref):
    for c in range(3):
        o_ref[:, c * H:(c + 1) * H] = (p_ref[c, 0] + p_ref[c, 1]
                                       - r_ref[:, c * H:(c + 1) * H])


def _combine_vec(pv, av384, blk=512):
    grid = (N + blk - 1) // blk
    return pl.pallas_call(
        _combv_body,
        grid=(grid,),
        in_specs=[pl.BlockSpec((3, 2, blk, H), lambda i: (0, 0, i, 0)),
                  pl.BlockSpec((blk, 3 * H), lambda i: (i, 0))],
        out_specs=pl.BlockSpec((blk, 3 * H), lambda i: (i, 0)),
        out_shape=jax.ShapeDtypeStruct((N, 3 * H), jnp.float32),
    )(pv, av384)


def _pad_idx(idx, total, pad_value):
    pad = total - idx.shape[0]
    p = jnp.concatenate([idx.astype(jnp.int32),
                         jnp.full((pad,), pad_value, jnp.int32)])
    return p.reshape(total // 128, 128)


_NW = 32


@functools.cache
def _sc_mesh():
    return plsc.VectorSubcoreMesh(core_axis_name="c", subcore_axis_name="s",
                                  num_cores=2, num_subcores=16)


def _sc_gather_all(a_xn, avF, m_xn, src2, src2_1, src2_2, dst2, asrc2, msrc2):
    nrows_e = src2.shape[0] * 128
    with_am = asrc2 is not None
    out_type = [jax.ShapeDtypeStruct((nrows_e, H), jnp.float32)
                for _ in range(5)]
    args = [a_xn, avF, m_xn, src2, src2_1, src2_2, dst2]
    if with_am:
        out_type += [jax.ShapeDtypeStruct((EAM_PAD, H), jnp.float32),
                     jax.ShapeDtypeStruct((EAM_PAD, H), jnp.float32)]
        args += [asrc2, msrc2]

    @functools.partial(
        pl.kernel,
        out_type=out_type,
        mesh=_sc_mesh(),
        scratch_types=[pltpu.VMEM((64, 128), jnp.int32),
                       pltpu.VMEM((128, H), jnp.float32),
                       pltpu.VMEM((128, H), jnp.float32),
                       pltpu.VMEM((128, H), jnp.float32),
                       pltpu.VMEM((128, H), jnp.float32),
                       pltpu.SemaphoreType.DMA,
                       pltpu.SemaphoreType.DMA,
                       pltpu.SemaphoreType.DMA,
                       pltpu.SemaphoreType.DMA],
    )
    def k(*refs):
        if with_am:
            (axn_hbm, avf_hbm, mxn_hbm,
             src_hbm, src1_hbm, src2_hbm, dst_hbm, asrc_hbm, msrc_hbm,
             gs_hbm, gd_hbm, gv0_hbm, gv1_hbm, gv2_hbm, ga_hbm, gm_hbm,
             idx_all, b0, b1, b2, b3, s0, s1, s2, s3) = refs
        else:
            (axn_hbm, avf_hbm, mxn_hbm,
             src_hbm, src1_hbm, src2_hbm, dst_hbm,
             gs_hbm, gd_hbm, gv0_hbm, gv1_hbm, gv2_hbm,
             idx_all, b0, b1, b2, b3, s0, s1, s2, s3) = refs
        cid = lax.axis_index("c")
        sid = lax.axis_index("s")
        bufs = (b0, b1, b2, b3)
        sems = (s0, s1, s2, s3)

        def gather_span(idx2, table, out, base, n):
            pltpu.sync_copy(idx2.at[pl.ds(base, n)], idx_all.at[pl.ds(0, n)])

            def body(j, carry):
                c0 = 4 * j
                ds = [pltpu.async_copy(table.at[idx_all.at[c0 + q]],
                                       bufs[q], sems[q]) for q in range(4)]
                for q in range(4):
                    ds[q].wait()
                    pltpu.sync_copy(
                        bufs[q], out.at[pl.ds((base + c0 + q) * 128, 128)])
                return carry
            lax.fori_loop(0, n // 4, body, 0)

        streams_e = ((src_hbm, axn_hbm, gs_hbm),
                     (dst_hbm, axn_hbm, gd_hbm),
                     (src_hbm, avf_hbm, gv0_hbm),
                     (src1_hbm, avf_hbm, gv1_hbm),
                     (src2_hbm, avf_hbm, gv2_hbm))
        streams_am = (((asrc_hbm, axn_hbm, ga_hbm),
                       (msrc_hbm, mxn_hbm, gm_hbm)) if with_am else ())

        RPW0_E, RPW1_E = 32, 8
        RPW0_AM, RPW1_AM = 32, 8

        @pl.when(cid == 0)
        def _():
            for idx2, table, out in streams_e:
                gather_span(idx2, table, out, sid * RPW0_E, RPW0_E)
            for idx2, table, out in streams_am:
                gather_span(idx2, table, out, sid * RPW0_AM, RPW0_AM)

        @pl.when(cid == 1)
        def _():
            for idx2, table, out in streams_e:
                gather_span(idx2, table, out, 16 * RPW0_E + sid * RPW1_E,
                            RPW1_E)
            for idx2, table, out in streams_am:
                gather_span(idx2, table, out, 16 * RPW0_AM + sid * RPW1_AM,
                            RPW1_AM)

    return k(*args)


def _sc_scatter_all(v1a, v1b, vm2a, va2m, v20a, v20b, v21a, v21b,
                    v22a, v22b, dsta3, dstb3, mdst3s, adst3s,
                    a_x, l_m, avT):
    @functools.partial(
        pl.kernel,
        out_type=[jax.ShapeDtypeStruct((N, H), jnp.float32),
                  jax.ShapeDtypeStruct((M, H), jnp.float32),
                  jax.ShapeDtypeStruct((3, N, H), jnp.float32)],
        mesh=_sc_mesh(),
        scratch_types=[pltpu.VMEM((EAM_PAD // 128 // 16, 128), jnp.int32),
                       pltpu.VMEM((128, H), jnp.float32),
                       pltpu.VMEM((128, H), jnp.float32),
                       pltpu.VMEM_SHARED((N_ACC, H), jnp.float32),
                       pltpu.SemaphoreType.DMA,
                       pltpu.SemaphoreType.DMA],
    )
    def k(v1a_hbm, v1b_hbm, vm2a_hbm, va2m_hbm, v20a_hbm, v20b_hbm,
          v21a_hbm, v21b_hbm, v22a_hbm, v22b_hbm,
          dsta_hbm, dstb_hbm, mdst_hbm, adst_hbm, ax_hbm, lm_hbm, avT_hbm,
          oax_hbm, omx_hbm, ovT_hbm,
          idx_all, val_a, val_b, shared, sem_a, sem_b):
        cid = lax.axis_index("c")
        sid = lax.axis_index("s")

        def sweep(nrows, do_chunk):
            nfull = nrows // 128
            tail = nrows - nfull * 128
            for r in range((nfull + 15) // 16):
                c_id = sid + r * 16
                @pl.when(c_id < nfull)
                def _():
                    do_chunk(pl.multiple_of(c_id * 128, 128), 128)
            if tail:
                @pl.when(sid == 15)
                def _():
                    do_chunk(nfull * 128, tail)

        def accum(vals, idx3, rpt):
            pltpu.sync_copy(idx3.at[sid], idx_all.at[pl.ds(0, rpt)])

            def body(j, carry):
                c0 = 2 * j
                r0 = sid * rpt + c0
                da = pltpu.async_copy(vals.at[pl.ds(r0 * 128, 128)],
                                      val_a, sem_a)
                db = pltpu.async_copy(vals.at[pl.ds((r0 + 1) * 128, 128)],
                                      val_b, sem_b)
                da.wait()
                pltpu.sync_copy(val_a, shared.at[idx_all.at[c0]], add=True)
                db.wait()
                pltpu.sync_copy(val_b, shared.at[idx_all.at[c0 + 1]],
                                add=True)
                return carry
            lax.fori_loop(0, rpt // 2, body, 0)

        def phase(init_src, streams, out_dst, nrows):
            def init_chunk(s, ch):
                pltpu.sync_copy(init_src(s, ch), val_a.at[pl.ds(0, ch)])
                pltpu.sync_copy(val_a.at[pl.ds(0, ch)], shared.at[pl.ds(s, ch)])
            sweep(nrows, init_chunk)
            plsc.subcore_barrier()
            for vals, idx3, rpt in streams:
                accum(vals, idx3, rpt)
            plsc.subcore_barrier()
            def out_chunk(s, ch):
                pltpu.sync_copy(shared.at[pl.ds(s, ch)], val_a.at[pl.ds(0, ch)])
                pltpu.sync_copy(val_a.at[pl.ds(0, ch)], out_dst(s, ch))
            sweep(nrows, out_chunk)
            plsc.subcore_barrier()

        rpt = EAM_PAD // 128 // 16

        @pl.when(cid == 0)
        def _():
            phase(lambda s, ch: ax_hbm.at[pl.ds(s, ch)],
                  [(v1a_hbm, dsta_hbm, rpt), (v1b_hbm, dstb_hbm, rpt),
                   (vm2a_hbm, mdst_hbm, rpt)],
                  lambda s, ch: oax_hbm.at[pl.ds(s, ch)], N)
            phase(lambda s, ch: avT_hbm.at[0, pl.ds(s, ch)],
                  [(v20a_hbm, dsta_hbm, rpt), (v20b_hbm, dstb_hbm, rpt)],
                  lambda s, ch: ovT_hbm.at[0, pl.ds(s, ch)], N)

        @pl.when(cid == 1)
        def _():
            phase(lambda s, ch: lm_hbm.at[pl.ds(s, ch)],
                  [(va2m_hbm, adst_hbm, rpt)],
                  lambda s, ch: omx_hbm.at[pl.ds(s, ch)], M)
            phase(lambda s, ch: avT_hbm.at[1, pl.ds(s, ch)],
                  [(v21a_hbm, dsta_hbm, rpt), (v21b_hbm, dstb_hbm, rpt)],
                  lambda s, ch: ovT_hbm.at[1, pl.ds(s, ch)], N)
            phase(lambda s, ch: avT_hbm.at[2, pl.ds(s, ch)],
                  [(v22a_hbm, dsta_hbm, rpt), (v22b_hbm, dstb_hbm, rpt)],
                  lambda s, ch: ovT_hbm.at[2, pl.ds(s, ch)], N)

    return k(v1a, v1b, vm2a, va2m, v20a, v20b, v21a, v21b, v22a, v22b,
             dsta3, dstb3, mdst3s, adst3s, a_x, l_m, avT)


def kernel(a_x, a_vec, m_x, a2a_edge_index, a2m_edge_index, m2a_edge_index,
           a2a_edge_weights, a2m_edge_weights, m2a_edge_weights,
           a2a_edge_attr, a2m_edge_attr, m2a_edge_attr, a2a_edge_vecs,
           W_s1, W_s2, W_e, W_a2m, W_m2a, Wq, Wk, Wv, Wo,
           ln_s_g, ln_s_b, ln_f_g, ln_f_b, ln_l_g, ln_l_b):
    a_xn = _ln(a_x, ln_s_g, ln_s_b)
    en = _ln(a2a_edge_attr, ln_f_g, ln_f_b, scale=1.0 / H)
    m_xn = _ln(m_x, ln_l_g, ln_l_b)

    src2 = _pad_idx(a2a_edge_index[0], E_PAD, 0)
    dstg2 = _pad_idx(a2a_edge_index[1], E_PAD, 0)
    dsts2 = _pad_idx(a2a_edge_index[1], E_PAD, N)
    asrc2 = _pad_idx(a2m_edge_index[0], EAM_PAD, 0)
    adst2 = _pad_idx(a2m_edge_index[1], EAM_PAD, M)
    msrc2 = _pad_idx(m2a_edge_index[0], EAM_PAD, 0)
    mdst2 = _pad_idx(m2a_edge_index[1], EAM_PAD, N)

    def s16(x):
        return x.reshape(16, -1, 128)

    avT = jnp.transpose(a_vec, (1, 0, 2))
    avF = avT.reshape(3 * N, H)

    HA = E_PAD // 2
    HROWS = HA // 128
    src2N, src22N = src2 + N, src2 + 2 * N
    gsA, gdA, gv0A, gv1A, gv2A, ga, gm = _sc_gather_all(
        a_xn, avF, m_xn, src2[:HROWS], src2N[:HROWS], src22N[:HROWS],
        dstg2[:HROWS], asrc2, msrc2)
    gsB, gdB, gv0B, gv1B, gv2B = _sc_gather_all(
        a_xn, avF, m_xn, src2[HROWS:], src2N[HROWS:], src22N[HROWS:],
        dstg2[HROWS:], None, None)

    w2d = a2a_edge_weights.reshape(E, 1)
    v1A, seA, v20A, v21A, v22A = _edge_math(
        gsA, gdA, gv0A, gv1A, gv2A, en, w2d, a2a_edge_vecs, a2a_edge_attr,
        W_s1, W_s2, W_e, 0, HA)
    v1B, seB, v20B, v21B, v22B = _edge_math(
        gsB, gdB, gv0B, gv1B, gv2B, en, w2d, a2a_edge_vecs, a2a_edge_attr,
        W_s1, W_s2, W_e, HA // 1024, E - HA)
    s_e_out = jnp.concatenate([seA, seB], axis=0)
    va2m = _pair_math(ga, a2m_edge_attr, a2m_edge_weights.reshape(EAM, 1),
                      W_a2m, EAM, EAM_PAD)
    vm2a = _pair_math(gm, m2a_edge_attr, m2a_edge_weights.reshape(EAM, 1),
                      W_m2a, EAM, EAM_PAD)

    l_m = _mha(m_xn, Wq, Wk, Wv, Wo, m_x)

    out_a_x, out_m_x, out_vT = _sc_scatter_all(
        v1A, v1B, vm2a, va2m, v20A, v20B, v21A, v21B, v22A, v22B,
        s16(dsts2[:HROWS]), s16(dsts2[HROWS:]), s16(mdst2), s16(adst2),
        a_x, l_m, avT)

    out_a_vec = jnp.transpose(out_vT, (1, 0, 2))
    return out_a_x, out_m_x, out_a_vec, s_e_out
